# baseline (speedup 1.0000x reference)
import functools

import jax
import jax.numpy as jnp
from jax import lax
from jax.experimental import pallas as pl
from jax.experimental.pallas import tpu as pltpu
from jax.experimental.pallas import tpu_sc as plsc

_N = 100000
_L = 256
_G = 512
_TILE = 10000
_NT = _N // _TILE
_W = 64
_NW = 32
_NP = 102400
_CH = _NP // _NW


def _pass1(lo_ref, hi_ref, z_ref, b_ref, w1_ref, b1_ref, w2t_ref, b2_ref,
           s_ref, e_ref, m_out_ref, m_ref):
    i = pl.program_id(0)

    @pl.when(i == 0)
    def _init():
        s_ref[:] = jnp.zeros_like(s_ref)
        m_ref[0, 0] = -jnp.inf

    zb = z_ref[:].astype(jnp.bfloat16)
    h = jnp.tanh(jnp.dot(zb, w1_ref[:].astype(jnp.bfloat16),
                         preferred_element_type=jnp.float32)
                 + b1_ref[:])
    a = jnp.sum(h * w2t_ref[:], axis=1, keepdims=True) + b2_ref[0, 0]
    m_ref[0, 0] = jnp.maximum(m_ref[0, 0], jnp.max(a))
    e = jnp.exp(a)
    e_ref[:] = e
    e_row = e.astype(jnp.bfloat16).reshape(1, _TILE)
    ids_row = b_ref[0]

    b0 = pl.multiple_of(jnp.minimum(lo_ref[i], _G - _W) & ~7, 8)

    @pl.when(hi_ref[i] - b0 < _W)
    def _narrow():
        oh = (jax.lax.broadcasted_iota(jnp.int32, (_W, _TILE), 0) + b0
              == ids_row).astype(jnp.bfloat16) * e_row
        s_ref[pl.ds(b0, _W), :] += jnp.dot(
            oh, zb, preferred_element_type=jnp.float32)

    @pl.when(hi_ref[i] - b0 >= _W)
    def _wide():
        oh = (jax.lax.broadcasted_iota(jnp.int32, (_G, _TILE), 0)
              == ids_row).astype(jnp.bfloat16) * e_row
        s_ref[:] += jnp.dot(oh, zb, preferred_element_type=jnp.float32)

    @pl.when(i == _NT - 1)
    def _final():
        m_out_ref[:] = jnp.full((1, 1), m_ref[0, 0], jnp.float32)


_sc_mesh = plsc.VectorSubcoreMesh(core_axis_name="c", subcore_axis_name="s")


@functools.partial(
    pl.kernel, mesh=_sc_mesh,
    out_type=jax.ShapeDtypeStruct((_NW, _G), jnp.float32),
    scratch_types=[
        pltpu.VMEM((_CH,), jnp.float32),
        pltpu.VMEM((_CH,), jnp.int32),
        pltpu.VMEM((_G,), jnp.float32),
        pltpu.SMEM((_G,), jnp.float32),
    ],
)
def _sc_segsum(e_hbm, ids_hbm, out_hbm, e_v, id_v, t_v, t_s):
    wid = lax.axis_index("s") * 2 + lax.axis_index("c")
    base = wid * _CH
    pltpu.sync_copy(e_hbm.at[pl.ds(base, _CH)], e_v)
    pltpu.sync_copy(ids_hbm.at[pl.ds(base, _CH)], id_v)

    lanes = lax.iota(jnp.int32, 16)

    def _zero(j, carry):
        t_s[j] = jnp.float32(0.0)
        return carry

    lax.fori_loop(0, _G, _zero, 0)

    def _scan(k, carry):
        e16 = e_v[pl.ds(k * 16, 16)]
        id16 = id_v[pl.ds(k * 16, 16)]
        for j in range(16):
            t_s[id16[j]] += e16[j]
        return carry

    lax.fori_loop(0, _CH // 16, _scan, 0)

    def _emit(j, carry):
        v = jnp.zeros((16,), jnp.float32)
        for jj in range(16):
            v = jnp.where(lanes == jj,
                          jnp.full((16,), t_s[j * 16 + jj], jnp.float32), v)
        t_v[pl.ds(j * 16, 16)] = v
        return carry

    lax.fori_loop(0, _G // 16, _emit, 0)
    pltpu.sync_copy(t_v, out_hbm.at[wid])


def _pass2(s_ref, tp_ref, m_ref, mw1_ref, mb1_ref, mw2_ref, mb2_ref,
           mw3t_ref, mb3_ref, out_ref):
    t_col = jnp.sum(tp_ref[:], axis=0, keepdims=True).reshape(_G, 1)
    eps = 1e-8 * jnp.exp(m_ref[0, 0])
    gz = s_ref[:] / (t_col + eps)
    x = jnp.maximum(jnp.dot(gz, mw1_ref[:],
                            preferred_element_type=jnp.float32)
                    + mb1_ref[:], 0.0)
    x = jnp.maximum(jnp.dot(x, mw2_ref[:],
                            preferred_element_type=jnp.float32)
                    + mb2_ref[:], 0.0)
    o = jnp.sum(x * mw3t_ref[:], axis=1, keepdims=True) + mb3_ref[0, 0]
    out_ref[:] = jax.nn.sigmoid(o)


def kernel(z, batch, att_w1, att_b1, att_w2, att_b2,
           mlp_w1, mlp_b1, mlp_w2, mlp_b2, mlp_w3, mlp_b3):
    b32 = batch.astype(jnp.int32)
    batch3d = b32.reshape(_NT, 1, _TILE)
    lo = b32[::_TILE]
    hi = b32[_TILE - 1::_TILE]
    full = lambda shape: pl.BlockSpec(shape, lambda i: (0, 0))
    smem = pl.BlockSpec(memory_space=pltpu.SMEM)
    s_out, e_out, m_out = pl.pallas_call(
        _pass1,
        grid=(_NT,),
        in_specs=[
            smem, smem,
            pl.BlockSpec((_TILE, _L), lambda i: (i, 0)),
            pl.BlockSpec((1, 1, _TILE), lambda i: (i, 0, 0)),
            full((_L, _L)),
            full((1, _L)),
            full((1, _L)),
            full((1, 1)),
        ],
        out_specs=[
            pl.BlockSpec((_G, _L), lambda i: (0, 0)),
            pl.BlockSpec((_TILE, 1), lambda i: (i, 0)),
            pl.BlockSpec((1, 1), lambda i: (0, 0)),
        ],
        out_shape=[
            jax.ShapeDtypeStruct((_G, _L), jnp.float32),
            jax.ShapeDtypeStruct((_N, 1), jnp.float32),
            jax.ShapeDtypeStruct((1, 1), jnp.float32),
        ],
        scratch_shapes=[pltpu.SMEM((1, 1), jnp.float32)],
    )(lo, hi, z, batch3d,
      att_w1, att_b1.reshape(1, _L), att_w2.reshape(1, _L),
      att_b2.reshape(1, 1))

    e_pad = jnp.concatenate([e_out.reshape(-1),
                             jnp.zeros((_NP - _N,), jnp.float32)])
    ids_pad = jnp.concatenate([b32, jnp.full((_NP - _N,), _G - 1, jnp.int32)])
    t_parts = _sc_segsum(e_pad, ids_pad)

    out2d = pl.pallas_call(
        _pass2,
        grid=(1,),
        in_specs=[
            full((_G, _L)),
            full((_NW, _G)),
            full((1, 1)),
            full((_L, 128)),
            full((1, 128)),
            full((128, 64)),
            full((1, 64)),
            full((1, 64)),
            full((1, 1)),
        ],
        out_specs=pl.BlockSpec((_G, 1), lambda i: (0, 0)),
        out_shape=jax.ShapeDtypeStruct((_G, 1), jnp.float32),
    )(s_out, t_parts, m_out,
      mlp_w1, mlp_b1.reshape(1, 128), mlp_w2, mlp_b2.reshape(1, 64),
      mlp_w3.reshape(1, 64), mlp_b3.reshape(1, 1))
    return out2d.reshape(-1)


# speedup vs baseline: 2.0269x; 2.0269x over previous
import jax
import jax.numpy as jnp
from jax.experimental import pallas as pl
from jax.experimental.pallas import tpu as pltpu

_N = 100000
_L = 256
_G = 512
_TILE = 10000
_NT = _N // _TILE
_W = 64


def _fused(lo_ref, hi_ref, z_ref, b_ref, w1_ref, b1_ref, w2t_ref, b2_ref,
           mw1_ref, mb1_ref, mw2_ref, mb2_ref, mw3t_ref, mb3_ref,
           out_ref, s_ref, t_ref, m_ref):
    i = pl.program_id(0)

    @pl.when(i == 0)
    def _init():
        s_ref[:] = jnp.zeros_like(s_ref)
        t_ref[:] = jnp.zeros_like(t_ref)
        m_ref[0, 0] = -jnp.inf

    zb = z_ref[:].astype(jnp.bfloat16)
    h = jnp.tanh(jnp.dot(zb, w1_ref[:].astype(jnp.bfloat16),
                         preferred_element_type=jnp.float32)
                 + b1_ref[:])
    a = jnp.sum(h * w2t_ref[:], axis=1, keepdims=True) + b2_ref[0, 0]
    m_ref[0, 0] = jnp.maximum(m_ref[0, 0], jnp.max(a))
    e_row = jnp.exp(a).astype(jnp.bfloat16).reshape(1, _TILE)
    ids_row = b_ref[0]
    ones = jnp.ones((_TILE, 1), jnp.bfloat16)

    b0 = pl.multiple_of(jnp.minimum(lo_ref[i], _G - _W) & ~7, 8)

    @pl.when(hi_ref[i] - b0 < _W)
    def _narrow():
        oh = (jax.lax.broadcasted_iota(jnp.int32, (_W, _TILE), 0) + b0
              == ids_row).astype(jnp.bfloat16) * e_row
        s_ref[pl.ds(b0, _W), :] += jnp.dot(
            oh, zb, preferred_element_type=jnp.float32)
        t_ref[pl.ds(b0, _W), :] += jnp.dot(
            oh, ones, preferred_element_type=jnp.float32)

    @pl.when(hi_ref[i] - b0 >= _W)
    def _wide():
        oh = (jax.lax.broadcasted_iota(jnp.int32, (_G, _TILE), 0)
              == ids_row).astype(jnp.bfloat16) * e_row
        s_ref[:] += jnp.dot(oh, zb, preferred_element_type=jnp.float32)
        t_ref[:] += jnp.dot(oh, ones, preferred_element_type=jnp.float32)

    @pl.when(i == _NT - 1)
    def _final():
        eps = 1e-8 * jnp.exp(m_ref[0, 0])
        gz = s_ref[:] / (t_ref[:] + eps)
        x = jnp.maximum(jnp.dot(gz, mw1_ref[:],
                                preferred_element_type=jnp.float32)
                        + mb1_ref[:], 0.0)
        x = jnp.maximum(jnp.dot(x, mw2_ref[:],
                                preferred_element_type=jnp.float32)
                        + mb2_ref[:], 0.0)
        o = jnp.sum(x * mw3t_ref[:], axis=1, keepdims=True) + mb3_ref[0, 0]
        out_ref[:] = jax.nn.sigmoid(o)


def kernel(z, batch, att_w1, att_b1, att_w2, att_b2,
           mlp_w1, mlp_b1, mlp_w2, mlp_b2, mlp_w3, mlp_b3):
    b32 = batch.astype(jnp.int32)
    batch3d = b32.reshape(_NT, 1, _TILE)
    lo = b32[::_TILE]
    hi = b32[_TILE - 1::_TILE]
    full = lambda shape: pl.BlockSpec(shape, lambda i: (0, 0))
    smem = pl.BlockSpec(memory_space=pltpu.SMEM)
    out2d = pl.pallas_call(
        _fused,
        grid=(_NT,),
        in_specs=[
            smem,
            smem,
            pl.BlockSpec((_TILE, _L), lambda i: (i, 0)),
            pl.BlockSpec((1, 1, _TILE), lambda i: (i, 0, 0)),
            full((_L, _L)),
            full((1, _L)),
            full((1, _L)),
            full((1, 1)),
            full((_L, 128)),
            full((1, 128)),
            full((128, 64)),
            full((1, 64)),
            full((1, 64)),
            full((1, 1)),
        ],
        out_specs=pl.BlockSpec((_G, 1), lambda i: (0, 0)),
        out_shape=jax.ShapeDtypeStruct((_G, 1), jnp.float32),
        scratch_shapes=[
            pltpu.VMEM((_G, _L), jnp.float32),
            pltpu.VMEM((_G, 1), jnp.float32),
            pltpu.SMEM((1, 1), jnp.float32),
        ],
    )(lo, hi, z, batch3d,
      att_w1, att_b1.reshape(1, _L), att_w2.reshape(1, _L),
      att_b2.reshape(1, 1),
      mlp_w1, mlp_b1.reshape(1, 128), mlp_w2, mlp_b2.reshape(1, 64),
      mlp_w3.reshape(1, 64), mlp_b3.reshape(1, 1))
    return out2d.reshape(-1)
